# baseline (speedup 1.0000x reference)
import functools

import jax
import jax.numpy as jnp
from jax import lax
from jax.experimental import pallas as pl
from jax.experimental.pallas import tpu as pltpu
from jax.experimental.pallas import tpu_sc as plsc

EMB_DIM = 64
VOCAB = 1000000
NUM_CORES = 2
NUM_SUBCORES = 16
NUM_WORKERS = NUM_CORES * NUM_SUBCORES

TSUB = 512
NSUB = 16
TW = TSUB * NSUB
TSTEPS = -(-VOCAB // TW)

CHUNK = 128
GROUP = 5
NGROUPS = 10


def _transpose_body(x_ref, o_ref, t_ref):
    for j in range(NSUB):
        blk = x_ref[:, pl.ds(j * TSUB, TSUB)]
        t_ref[...] = blk.T
        r = pl.ds(j * TSUB // 2, TSUB // 2)
        o_ref[r, pl.ds(0, EMB_DIM)] = t_ref[pl.Slice(0, TSUB // 2, 2), :]
        o_ref[r, pl.ds(EMB_DIM, EMB_DIM)] = t_ref[pl.Slice(1, TSUB // 2, 2), :]


@jax.jit
def _lookup(idx3, table_t):
    scratch = pl.pallas_call(
        _transpose_body,
        grid=(TSTEPS,),
        in_specs=[pl.BlockSpec((EMB_DIM, TW), lambda i: (0, i))],
        out_specs=pl.BlockSpec((TW // 2, 2 * EMB_DIM), lambda i: (i, 0)),
        out_shape=jax.ShapeDtypeStruct(
            (TSTEPS * TW // 2, 2 * EMB_DIM), jnp.float32
        ),
        scratch_shapes=[pltpu.VMEM((TSUB, EMB_DIM), jnp.float32)],
    )(table_t)
    rows = scratch.reshape(TSTEPS * TW, EMB_DIM)

    n_ch = GROUP * NGROUPS
    b_per_w = n_ch * CHUNK
    B = NUM_WORKERS * b_per_w
    grp_rows = GROUP * CHUNK
    mesh = plsc.VectorSubcoreMesh(core_axis_name="c", subcore_axis_name="s")

    @functools.partial(
        pl.kernel,
        mesh=mesh,
        out_type=jax.ShapeDtypeStruct((B, EMB_DIM), jnp.float32),
        compiler_params=pltpu.CompilerParams(use_tc_tiling_on_sc=False),
        scratch_types=[
            pltpu.VMEM((n_ch, CHUNK), jnp.int32),
            pltpu.VMEM((grp_rows, EMB_DIM), jnp.float32),
            pltpu.VMEM((grp_rows, EMB_DIM), jnp.float32),
            pltpu.SemaphoreType.DMA,
            pltpu.SemaphoreType.DMA,
        ],
    )
    def kgather(table_hbm, idx_hbm, out_hbm, idx_v, buf_a, buf_b, sem_a, sem_b):
        wid = lax.axis_index("s") * NUM_CORES + lax.axis_index("c")
        base = wid * b_per_w
        pltpu.sync_copy(idx_hbm.at[wid], idx_v)

        def fire(g, buf, sem):
            for j in range(GROUP):
                pltpu.make_async_copy(
                    table_hbm.at[idx_v.at[g * GROUP + j]],
                    buf.at[pl.ds(j * CHUNK, CHUNK)],
                    sem,
                ).start()

        def drain(g, buf, sem):
            for j in range(GROUP):
                pltpu.make_async_copy(
                    table_hbm.at[idx_v.at[g * GROUP + j]],
                    buf.at[pl.ds(j * CHUNK, CHUNK)],
                    sem,
                ).wait()
            pltpu.sync_copy(buf, out_hbm.at[pl.ds(base + g * grp_rows, grp_rows)])

        fire(0, buf_a, sem_a)

        @pl.loop(0, NGROUPS, step=2)
        def _(g):
            @pl.when(g + 1 < NGROUPS)
            def _():
                fire(g + 1, buf_b, sem_b)

            drain(g, buf_a, sem_a)

            @pl.when(g + 2 < NGROUPS)
            def _():
                fire(g + 2, buf_a, sem_a)

            @pl.when(g + 1 < NGROUPS)
            def _():
                drain(g + 1, buf_b, sem_b)

    return kgather(rows, idx3)


def kernel(x, table):
    B = x.shape[0] * x.shape[1]
    n_ch = GROUP * NGROUPS
    idx3 = x.reshape(NUM_WORKERS, n_ch, CHUNK)
    out = _lookup(idx3, table.T)
    return out.reshape(x.shape[0], x.shape[1], EMB_DIM)


# speedup vs baseline: 1.0518x; 1.0518x over previous
import functools

import jax
import jax.numpy as jnp
from jax import lax
from jax.experimental import pallas as pl
from jax.experimental.pallas import tpu as pltpu
from jax.experimental.pallas import tpu_sc as plsc

EMB_DIM = 64
VOCAB = 1000000
NUM_CORES = 2
NUM_SUBCORES = 16
NUM_WORKERS = NUM_CORES * NUM_SUBCORES

TSUB = 512
NSUB = 16
TW = TSUB * NSUB
TSTEPS = -(-VOCAB // TW)

CHUNK = 128
GROUP = 5
NGROUPS = 10


def _transpose_body(x_ref, o_ref, t_ref):
    for j in range(NSUB):
        t_ref[pl.ds(j * TSUB, TSUB), :] = x_ref[:, pl.ds(j * TSUB, TSUB)].T
    ev = t_ref[pl.Slice(0, TW // 2, 2), :]
    od = t_ref[pl.Slice(1, TW // 2, 2), :]
    o_ref[...] = jnp.concatenate([ev, od], axis=1)


@jax.jit
def _lookup(idx3, table_t):
    scratch = pl.pallas_call(
        _transpose_body,
        grid=(TSTEPS,),
        in_specs=[pl.BlockSpec((EMB_DIM, TW), lambda i: (0, i))],
        out_specs=pl.BlockSpec((TW // 2, 2 * EMB_DIM), lambda i: (i, 0)),
        out_shape=jax.ShapeDtypeStruct(
            (TSTEPS * TW // 2, 2 * EMB_DIM), jnp.float32
        ),
        scratch_shapes=[pltpu.VMEM((TW, EMB_DIM), jnp.float32)],
    )(table_t)
    rows = scratch.reshape(TSTEPS * TW, EMB_DIM)

    n_ch = GROUP * NGROUPS
    b_per_w = n_ch * CHUNK
    B = NUM_WORKERS * b_per_w
    grp_rows = GROUP * CHUNK
    mesh = plsc.VectorSubcoreMesh(core_axis_name="c", subcore_axis_name="s")

    @functools.partial(
        pl.kernel,
        mesh=mesh,
        out_type=jax.ShapeDtypeStruct((B, EMB_DIM), jnp.float32),
        compiler_params=pltpu.CompilerParams(use_tc_tiling_on_sc=False),
        scratch_types=[
            pltpu.VMEM((n_ch, CHUNK), jnp.int32),
            pltpu.VMEM((grp_rows, EMB_DIM), jnp.float32),
            pltpu.VMEM((grp_rows, EMB_DIM), jnp.float32),
            pltpu.SemaphoreType.DMA,
            pltpu.SemaphoreType.DMA,
        ],
    )
    def kgather(table_hbm, idx_hbm, out_hbm, idx_v, buf_a, buf_b, sem_a, sem_b):
        wid = lax.axis_index("s") * NUM_CORES + lax.axis_index("c")
        base = wid * b_per_w
        pltpu.sync_copy(idx_hbm.at[wid], idx_v)

        def fire(g, buf, sem):
            for j in range(GROUP):
                pltpu.make_async_copy(
                    table_hbm.at[idx_v.at[g * GROUP + j]],
                    buf.at[pl.ds(j * CHUNK, CHUNK)],
                    sem,
                ).start()

        def drain(g, buf, sem):
            for j in range(GROUP):
                pltpu.make_async_copy(
                    table_hbm.at[idx_v.at[g * GROUP + j]],
                    buf.at[pl.ds(j * CHUNK, CHUNK)],
                    sem,
                ).wait()
            pltpu.sync_copy(buf, out_hbm.at[pl.ds(base + g * grp_rows, grp_rows)])

        fire(0, buf_a, sem_a)

        @pl.loop(0, NGROUPS, step=2)
        def _(g):
            @pl.when(g + 1 < NGROUPS)
            def _():
                fire(g + 1, buf_b, sem_b)

            drain(g, buf_a, sem_a)

            @pl.when(g + 2 < NGROUPS)
            def _():
                fire(g + 2, buf_a, sem_a)

            @pl.when(g + 1 < NGROUPS)
            def _():
                drain(g + 1, buf_b, sem_b)

    return kgather(rows, idx3)


def kernel(x, table):
    B = x.shape[0] * x.shape[1]
    n_ch = GROUP * NGROUPS
    idx3 = x.reshape(NUM_WORKERS, n_ch, CHUNK)
    out = _lookup(idx3, table.T)
    return out.reshape(x.shape[0], x.shape[1], EMB_DIM)


# speedup vs baseline: 1.0532x; 1.0013x over previous
import functools

import jax
import jax.numpy as jnp
from jax import lax
from jax.experimental import pallas as pl
from jax.experimental.pallas import tpu as pltpu
from jax.experimental.pallas import tpu_sc as plsc

EMB_DIM = 64
VOCAB = 1000000
NUM_CORES = 2
NUM_SUBCORES = 16
NUM_WORKERS = NUM_CORES * NUM_SUBCORES

TSUB = 512
NSUB = 16
TW = TSUB * NSUB
TSTEPS = -(-VOCAB // TW)

CHUNK = 128
GROUP = 5
NGROUPS = 10


def _transpose_body(x_ref, o_ref, t_ref):
    for j in range(NSUB):
        t_ref[pl.ds(j * TSUB, TSUB), :] = x_ref[:, pl.ds(j * TSUB, TSUB)].T
    ev = t_ref[pl.Slice(0, TW // 2, 2), :]
    od = t_ref[pl.Slice(1, TW // 2, 2), :]
    o_ref[...] = jnp.concatenate([ev, od], axis=1)


@jax.jit
def _lookup(idx3, table_t):
    scratch = pl.pallas_call(
        _transpose_body,
        grid=(TSTEPS,),
        in_specs=[pl.BlockSpec((EMB_DIM, TW), lambda i: (0, i))],
        out_specs=pl.BlockSpec((TW // 2, 2 * EMB_DIM), lambda i: (i, 0)),
        out_shape=jax.ShapeDtypeStruct(
            (TSTEPS * TW // 2, 2 * EMB_DIM), jnp.float32
        ),
        scratch_shapes=[pltpu.VMEM((TW, EMB_DIM), jnp.float32)],
        compiler_params=pltpu.CompilerParams(
            dimension_semantics=("arbitrary",)
        ),
    )(table_t)
    rows = scratch.reshape(TSTEPS * TW, EMB_DIM)

    n_ch = GROUP * NGROUPS
    b_per_w = n_ch * CHUNK
    B = NUM_WORKERS * b_per_w
    grp_rows = GROUP * CHUNK
    mesh = plsc.VectorSubcoreMesh(core_axis_name="c", subcore_axis_name="s")

    @functools.partial(
        pl.kernel,
        mesh=mesh,
        out_type=jax.ShapeDtypeStruct((B, EMB_DIM), jnp.float32),
        compiler_params=pltpu.CompilerParams(use_tc_tiling_on_sc=False),
        scratch_types=[
            pltpu.VMEM((n_ch, CHUNK), jnp.int32),
            pltpu.VMEM((grp_rows, EMB_DIM), jnp.float32),
            pltpu.VMEM((grp_rows, EMB_DIM), jnp.float32),
            pltpu.SemaphoreType.DMA,
            pltpu.SemaphoreType.DMA,
        ],
    )
    def kgather(table_hbm, idx_hbm, out_hbm, idx_v, buf_a, buf_b, sem_a, sem_b):
        wid = lax.axis_index("s") * NUM_CORES + lax.axis_index("c")
        base = wid * b_per_w
        pltpu.sync_copy(idx_hbm.at[wid], idx_v)

        def fire(g, buf, sem):
            for j in range(GROUP):
                pltpu.make_async_copy(
                    table_hbm.at[idx_v.at[g * GROUP + j]],
                    buf.at[pl.ds(j * CHUNK, CHUNK)],
                    sem,
                ).start()

        def drain(g, buf, sem):
            for j in range(GROUP):
                pltpu.make_async_copy(
                    table_hbm.at[idx_v.at[g * GROUP + j]],
                    buf.at[pl.ds(j * CHUNK, CHUNK)],
                    sem,
                ).wait()
            pltpu.sync_copy(buf, out_hbm.at[pl.ds(base + g * grp_rows, grp_rows)])

        fire(0, buf_a, sem_a)

        @pl.loop(0, NGROUPS, step=2)
        def _(g):
            @pl.when(g + 1 < NGROUPS)
            def _():
                fire(g + 1, buf_b, sem_b)

            drain(g, buf_a, sem_a)

            @pl.when(g + 2 < NGROUPS)
            def _():
                fire(g + 2, buf_a, sem_a)

            @pl.when(g + 1 < NGROUPS)
            def _():
                drain(g + 1, buf_b, sem_b)

    return kgather(rows, idx3)


def kernel(x, table):
    B = x.shape[0] * x.shape[1]
    n_ch = GROUP * NGROUPS
    idx3 = x.reshape(NUM_WORKERS, n_ch, CHUNK)
    out = _lookup(idx3, table.T)
    return out.reshape(x.shape[0], x.shape[1], EMB_DIM)


# speedup vs baseline: 1.1395x; 1.0819x over previous
import functools

import jax
import jax.numpy as jnp
from jax import lax
from jax.experimental import pallas as pl
from jax.experimental.pallas import tpu as pltpu
from jax.experimental.pallas import tpu_sc as plsc

EMB_DIM = 64
VOCAB = 1000000
NUM_CORES = 2
NUM_SUBCORES = 16
NUM_WORKERS = NUM_CORES * NUM_SUBCORES

TSUB = 1024
NSUB = 16
TW = TSUB * NSUB
TSTEPS = -(-VOCAB // TW)

CHUNK = 128
GROUP = 5
NGROUPS = 10


def _transpose_body(x_ref, o_ref, t_ref):
    for j in range(NSUB):
        t_ref[pl.ds(j * TSUB, TSUB), :] = x_ref[:, pl.ds(j * TSUB, TSUB)].T
    ev = t_ref[pl.Slice(0, TW // 2, 2), :]
    od = t_ref[pl.Slice(1, TW // 2, 2), :]
    o_ref[...] = jnp.concatenate([ev, od], axis=1)


@jax.jit
def _lookup(idx3, table_t):
    scratch = pl.pallas_call(
        _transpose_body,
        grid=(TSTEPS,),
        in_specs=[pl.BlockSpec((EMB_DIM, TW), lambda i: (0, i))],
        out_specs=pl.BlockSpec((TW // 2, 2 * EMB_DIM), lambda i: (i, 0)),
        out_shape=jax.ShapeDtypeStruct(
            (TSTEPS * TW // 2, 2 * EMB_DIM), jnp.float32
        ),
        scratch_shapes=[pltpu.VMEM((TW, EMB_DIM), jnp.float32)],
        compiler_params=pltpu.CompilerParams(
            dimension_semantics=("arbitrary",)
        ),
    )(table_t)
    rows = scratch.reshape(TSTEPS * TW, EMB_DIM)

    n_ch = GROUP * NGROUPS
    b_per_w = n_ch * CHUNK
    B = NUM_WORKERS * b_per_w
    grp_rows = GROUP * CHUNK
    mesh = plsc.VectorSubcoreMesh(core_axis_name="c", subcore_axis_name="s")

    @functools.partial(
        pl.kernel,
        mesh=mesh,
        out_type=jax.ShapeDtypeStruct((B, EMB_DIM), jnp.float32),
        compiler_params=pltpu.CompilerParams(use_tc_tiling_on_sc=False),
        scratch_types=[
            pltpu.VMEM((n_ch, CHUNK), jnp.int32),
            pltpu.VMEM((grp_rows, EMB_DIM), jnp.float32),
            pltpu.VMEM((grp_rows, EMB_DIM), jnp.float32),
            pltpu.SemaphoreType.DMA,
            pltpu.SemaphoreType.DMA,
        ],
    )
    def kgather(table_hbm, idx_hbm, out_hbm, idx_v, buf_a, buf_b, sem_a, sem_b):
        wid = lax.axis_index("s") * NUM_CORES + lax.axis_index("c")
        base = wid * b_per_w
        pltpu.sync_copy(idx_hbm.at[wid], idx_v)

        def fire(g, buf, sem):
            for j in range(GROUP):
                pltpu.make_async_copy(
                    table_hbm.at[idx_v.at[g * GROUP + j]],
                    buf.at[pl.ds(j * CHUNK, CHUNK)],
                    sem,
                ).start()

        def drain(g, buf, sem):
            for j in range(GROUP):
                pltpu.make_async_copy(
                    table_hbm.at[idx_v.at[g * GROUP + j]],
                    buf.at[pl.ds(j * CHUNK, CHUNK)],
                    sem,
                ).wait()
            pltpu.sync_copy(buf, out_hbm.at[pl.ds(base + g * grp_rows, grp_rows)])

        fire(0, buf_a, sem_a)

        @pl.loop(0, NGROUPS, step=2)
        def _(g):
            @pl.when(g + 1 < NGROUPS)
            def _():
                fire(g + 1, buf_b, sem_b)

            drain(g, buf_a, sem_a)

            @pl.when(g + 2 < NGROUPS)
            def _():
                fire(g + 2, buf_a, sem_a)

            @pl.when(g + 1 < NGROUPS)
            def _():
                drain(g + 1, buf_b, sem_b)

    return kgather(rows, idx3)


def kernel(x, table):
    B = x.shape[0] * x.shape[1]
    n_ch = GROUP * NGROUPS
    idx3 = x.reshape(NUM_WORKERS, n_ch, CHUNK)
    out = _lookup(idx3, table.T)
    return out.reshape(x.shape[0], x.shape[1], EMB_DIM)


# speedup vs baseline: 1.1801x; 1.0356x over previous
import functools

import jax
import jax.numpy as jnp
from jax import lax
from jax.experimental import pallas as pl
from jax.experimental.pallas import tpu as pltpu
from jax.experimental.pallas import tpu_sc as plsc

EMB_DIM = 64
VOCAB = 1000000
NUM_CORES = 2
NUM_SUBCORES = 16
NUM_WORKERS = NUM_CORES * NUM_SUBCORES

TSUB = 1024
NSUB = 32
TW = TSUB * NSUB
TSTEPS = -(-VOCAB // TW)

CHUNK = 128
GROUP = 5
NGROUPS = 10


def _transpose_body(x_ref, o_ref, t_ref):
    for j in range(NSUB):
        t_ref[pl.ds(j * TSUB, TSUB), :] = x_ref[:, pl.ds(j * TSUB, TSUB)].T
    ev = t_ref[pl.Slice(0, TW // 2, 2), :]
    od = t_ref[pl.Slice(1, TW // 2, 2), :]
    o_ref[...] = jnp.concatenate([ev, od], axis=1)


@jax.jit
def _lookup(idx3, table_t):
    scratch = pl.pallas_call(
        _transpose_body,
        grid=(TSTEPS,),
        in_specs=[pl.BlockSpec((EMB_DIM, TW), lambda i: (0, i))],
        out_specs=pl.BlockSpec((TW // 2, 2 * EMB_DIM), lambda i: (i, 0)),
        out_shape=jax.ShapeDtypeStruct(
            (TSTEPS * TW // 2, 2 * EMB_DIM), jnp.float32
        ),
        scratch_shapes=[pltpu.VMEM((TW, EMB_DIM), jnp.float32)],
        compiler_params=pltpu.CompilerParams(
            dimension_semantics=("arbitrary",)
        ),
    )(table_t)
    rows = scratch.reshape(TSTEPS * TW, EMB_DIM)

    n_ch = GROUP * NGROUPS
    b_per_w = n_ch * CHUNK
    B = NUM_WORKERS * b_per_w
    grp_rows = GROUP * CHUNK
    mesh = plsc.VectorSubcoreMesh(core_axis_name="c", subcore_axis_name="s")

    @functools.partial(
        pl.kernel,
        mesh=mesh,
        out_type=jax.ShapeDtypeStruct((B, EMB_DIM), jnp.float32),
        compiler_params=pltpu.CompilerParams(use_tc_tiling_on_sc=False),
        scratch_types=[
            pltpu.VMEM((n_ch, CHUNK), jnp.int32),
            pltpu.VMEM((grp_rows, EMB_DIM), jnp.float32),
            pltpu.VMEM((grp_rows, EMB_DIM), jnp.float32),
            pltpu.SemaphoreType.DMA,
            pltpu.SemaphoreType.DMA,
        ],
    )
    def kgather(table_hbm, idx_hbm, out_hbm, idx_v, buf_a, buf_b, sem_a, sem_b):
        wid = lax.axis_index("s") * NUM_CORES + lax.axis_index("c")
        base = wid * b_per_w
        pltpu.sync_copy(idx_hbm.at[wid], idx_v)

        def fire(g, buf, sem):
            for j in range(GROUP):
                pltpu.make_async_copy(
                    table_hbm.at[idx_v.at[g * GROUP + j]],
                    buf.at[pl.ds(j * CHUNK, CHUNK)],
                    sem,
                ).start()

        def drain(g, buf, sem):
            for j in range(GROUP):
                pltpu.make_async_copy(
                    table_hbm.at[idx_v.at[g * GROUP + j]],
                    buf.at[pl.ds(j * CHUNK, CHUNK)],
                    sem,
                ).wait()
            pltpu.sync_copy(buf, out_hbm.at[pl.ds(base + g * grp_rows, grp_rows)])

        fire(0, buf_a, sem_a)

        @pl.loop(0, NGROUPS, step=2)
        def _(g):
            @pl.when(g + 1 < NGROUPS)
            def _():
                fire(g + 1, buf_b, sem_b)

            drain(g, buf_a, sem_a)

            @pl.when(g + 2 < NGROUPS)
            def _():
                fire(g + 2, buf_a, sem_a)

            @pl.when(g + 1 < NGROUPS)
            def _():
                drain(g + 1, buf_b, sem_b)

    return kgather(rows, idx3)


def kernel(x, table):
    B = x.shape[0] * x.shape[1]
    n_ch = GROUP * NGROUPS
    idx3 = x.reshape(NUM_WORKERS, n_ch, CHUNK)
    out = _lookup(idx3, table.T)
    return out.reshape(x.shape[0], x.shape[1], EMB_DIM)


# speedup vs baseline: 1.1819x; 1.0016x over previous
import functools

import jax
import jax.numpy as jnp
from jax import lax
from jax.experimental import pallas as pl
from jax.experimental.pallas import tpu as pltpu
from jax.experimental.pallas import tpu_sc as plsc

EMB_DIM = 64
VOCAB = 1000000
NUM_CORES = 2
NUM_SUBCORES = 16
NUM_WORKERS = NUM_CORES * NUM_SUBCORES

TSUB = 2048
NSUB = 16
TW = TSUB * NSUB
TSTEPS = -(-VOCAB // TW)

CHUNK = 128
GROUP = 5
NGROUPS = 10


def _transpose_body(x_ref, o_ref, t_ref):
    for j in range(NSUB):
        t_ref[...] = x_ref[:, pl.ds(j * TSUB, TSUB)].T
        ev = t_ref[pl.Slice(0, TSUB // 2, 2), :]
        od = t_ref[pl.Slice(1, TSUB // 2, 2), :]
        o_ref[pl.ds(j * TSUB // 2, TSUB // 2), :] = jnp.concatenate(
            [ev, od], axis=1
        )


@jax.jit
def _lookup(idx3, table_t):
    scratch = pl.pallas_call(
        _transpose_body,
        grid=(TSTEPS,),
        in_specs=[pl.BlockSpec((EMB_DIM, TW), lambda i: (0, i))],
        out_specs=pl.BlockSpec((TW // 2, 2 * EMB_DIM), lambda i: (i, 0)),
        out_shape=jax.ShapeDtypeStruct(
            (TSTEPS * TW // 2, 2 * EMB_DIM), jnp.float32
        ),
        scratch_shapes=[pltpu.VMEM((TSUB, EMB_DIM), jnp.float32)],
        compiler_params=pltpu.CompilerParams(
            dimension_semantics=("arbitrary",)
        ),
    )(table_t)
    rows = scratch.reshape(TSTEPS * TW, EMB_DIM)

    n_ch = GROUP * NGROUPS
    b_per_w = n_ch * CHUNK
    B = NUM_WORKERS * b_per_w
    grp_rows = GROUP * CHUNK
    mesh = plsc.VectorSubcoreMesh(core_axis_name="c", subcore_axis_name="s")

    @functools.partial(
        pl.kernel,
        mesh=mesh,
        out_type=jax.ShapeDtypeStruct((B, EMB_DIM), jnp.float32),
        compiler_params=pltpu.CompilerParams(use_tc_tiling_on_sc=False),
        scratch_types=[
            pltpu.VMEM((n_ch, CHUNK), jnp.int32),
            pltpu.VMEM((grp_rows, EMB_DIM), jnp.float32),
            pltpu.VMEM((grp_rows, EMB_DIM), jnp.float32),
            pltpu.SemaphoreType.DMA,
            pltpu.SemaphoreType.DMA,
        ],
    )
    def kgather(table_hbm, idx_hbm, out_hbm, idx_v, buf_a, buf_b, sem_a, sem_b):
        wid = lax.axis_index("s") * NUM_CORES + lax.axis_index("c")
        base = wid * b_per_w
        pltpu.sync_copy(idx_hbm.at[wid], idx_v)

        def fire(g, buf, sem):
            for j in range(GROUP):
                pltpu.make_async_copy(
                    table_hbm.at[idx_v.at[g * GROUP + j]],
                    buf.at[pl.ds(j * CHUNK, CHUNK)],
                    sem,
                ).start()

        def drain(g, buf, sem):
            for j in range(GROUP):
                pltpu.make_async_copy(
                    table_hbm.at[idx_v.at[g * GROUP + j]],
                    buf.at[pl.ds(j * CHUNK, CHUNK)],
                    sem,
                ).wait()
            pltpu.sync_copy(buf, out_hbm.at[pl.ds(base + g * grp_rows, grp_rows)])

        fire(0, buf_a, sem_a)

        @pl.loop(0, NGROUPS, step=2)
        def _(g):
            @pl.when(g + 1 < NGROUPS)
            def _():
                fire(g + 1, buf_b, sem_b)

            drain(g, buf_a, sem_a)

            @pl.when(g + 2 < NGROUPS)
            def _():
                fire(g + 2, buf_a, sem_a)

            @pl.when(g + 1 < NGROUPS)
            def _():
                drain(g + 1, buf_b, sem_b)

    return kgather(rows, idx3)


def kernel(x, table):
    B = x.shape[0] * x.shape[1]
    n_ch = GROUP * NGROUPS
    idx3 = x.reshape(NUM_WORKERS, n_ch, CHUNK)
    out = _lookup(idx3, table.T)
    return out.reshape(x.shape[0], x.shape[1], EMB_DIM)
